# Initial kernel scaffold; baseline (speedup 1.0000x reference)
import functools

import jax
import jax.numpy as jnp
from jax.experimental import pallas as pl
import jax.experimental.pallas.tpu as pltpu


def _body(idx_sref, tgt_ref, *rest, n_steps, n_rows, G, C):
    table_refs = rest[:G]
    out_ref, loss_ref, acc = rest[G], rest[G + 1], rest[G + 2]
    step = pl.program_id(0)

    @pl.when(step == 0)
    def _init():
        acc[0] = jnp.float32(0.0)

    rows = jnp.concatenate([r[...] for r in table_refs], axis=0)
    out_ref[...] = rows

    m = jnp.max(rows, axis=1, keepdims=True)
    e = jnp.exp(rows - m)
    s = jnp.sum(e, axis=1, keepdims=True)
    lse = m + jnp.log(s)

    tgt = tgt_ref[0, 0, :]
    lane = jax.lax.broadcasted_iota(jnp.int32, (G, C), 1)
    tv = jnp.sum(jnp.where(lane == tgt[:, None], rows, 0.0), axis=1,
                 keepdims=True)
    acc[0] += jnp.sum(lse - tv)

    @pl.when(step == n_steps - 1)
    def _fin():
        loss_ref[0, 0] = acc[0] / jnp.float32(n_rows)


def kernel(idx, targets, table):
    B, T = idx.shape
    V, C = table.shape
    N = B * T
    G = 16
    n_steps = N // G

    idx_flat = idx.reshape(N).astype(jnp.int32)
    tgt3 = targets.reshape(n_steps, 1, G).astype(jnp.int32)

    grid_spec = pltpu.PrefetchScalarGridSpec(
        num_scalar_prefetch=1,
        grid=(n_steps,),
        in_specs=(
            [pl.BlockSpec((1, 1, G), lambda i, idx_ref: (i, 0, 0))]
            + [
                pl.BlockSpec(
                    (1, C),
                    functools.partial(
                        lambda j, i, idx_ref: (idx_ref[i * G + j], 0), j
                    ),
                )
                for j in range(G)
            ]
        ),
        out_specs=[
            pl.BlockSpec((G, C), lambda i, idx_ref: (i, 0)),
            pl.BlockSpec((1, 1), lambda i, idx_ref: (0, 0)),
        ],
        scratch_shapes=[pltpu.SMEM((1,), jnp.float32)],
    )

    logits, loss = pl.pallas_call(
        functools.partial(_body, n_steps=n_steps, n_rows=N, G=G, C=C),
        grid_spec=grid_spec,
        out_shape=[
            jax.ShapeDtypeStruct((N, C), jnp.float32),
            jax.ShapeDtypeStruct((1, 1), jnp.float32),
        ],
    )(idx_flat, tgt3, *([table] * G))
    return (logits, loss[0, 0])


# speedup vs baseline: 1.1967x; 1.1967x over previous
import functools

import jax
import jax.numpy as jnp
from jax.experimental import pallas as pl
import jax.experimental.pallas.tpu as pltpu


def _body(idx_sref, tgt_ref, *rest, n_steps, n_rows, G, C):
    table_refs = rest[:G]
    out_ref, loss_ref, acc = rest[G], rest[G + 1], rest[G + 2]
    step = pl.program_id(0)

    @pl.when(step == 0)
    def _init():
        acc[0] = jnp.float32(0.0)

    rows = jnp.concatenate([r[0] for r in table_refs], axis=0)
    out_ref[...] = rows

    m = jnp.max(rows, axis=1, keepdims=True)
    e = jnp.exp(rows - m)
    s = jnp.sum(e, axis=1, keepdims=True)
    lse = m + jnp.log(s)

    tgt = tgt_ref[0, 0, :]
    lane = jax.lax.broadcasted_iota(jnp.int32, (G, C), 1)
    tv = jnp.sum(jnp.where(lane == tgt[:, None], rows, 0.0), axis=1,
                 keepdims=True)
    acc[0] += jnp.sum(lse - tv)

    @pl.when(step == n_steps - 1)
    def _fin():
        loss_ref[...] = jnp.full((1, 1), acc[0] / jnp.float32(n_rows),
                                 dtype=jnp.float32)


def kernel(idx, targets, table):
    B, T = idx.shape
    V, C = table.shape
    N = B * T
    G = 32
    n_steps = N // G

    idx_flat = idx.reshape(N).astype(jnp.int32)
    tgt3 = targets.reshape(n_steps, 1, G).astype(jnp.int32)
    table3 = table.reshape(V, 1, C)

    grid_spec = pltpu.PrefetchScalarGridSpec(
        num_scalar_prefetch=1,
        grid=(n_steps,),
        in_specs=(
            [pl.BlockSpec((1, 1, G), lambda i, idx_ref: (i, 0, 0))]
            + [
                pl.BlockSpec(
                    (1, 1, C),
                    functools.partial(
                        lambda j, i, idx_ref: (idx_ref[i * G + j], 0, 0), j
                    ),
                )
                for j in range(G)
            ]
        ),
        out_specs=[
            pl.BlockSpec((G, C), lambda i, idx_ref: (i, 0)),
            pl.BlockSpec((1, 1), lambda i, idx_ref: (0, 0)),
        ],
        scratch_shapes=[pltpu.SMEM((1,), jnp.float32)],
    )

    logits, loss = pl.pallas_call(
        functools.partial(_body, n_steps=n_steps, n_rows=N, G=G, C=C),
        grid_spec=grid_spec,
        out_shape=[
            jax.ShapeDtypeStruct((N, C), jnp.float32),
            jax.ShapeDtypeStruct((1, 1), jnp.float32),
        ],
    )(idx_flat, tgt3, *([table3] * G))
    return (logits, loss[0, 0])


# speedup vs baseline: 2.0332x; 1.6991x over previous
import functools

import jax
import jax.numpy as jnp
from jax import lax
from jax.experimental import pallas as pl
from jax.experimental.pallas import tpu as pltpu
from jax.experimental.pallas import tpu_sc as plsc

_NC, _NS = 2, 16
_NW = _NC * _NS
_NBUF = 2


def _sc_gather_body(idx_hbm, table_hbm, out_hbm, idx_v, bufs, gsems, ssems,
                    *, rows_per_w, chunk):
    wid = lax.axis_index("s") * _NC + lax.axis_index("c")
    base = wid * rows_per_w
    pltpu.sync_copy(idx_hbm.at[wid], idx_v)
    n_iter = rows_per_w // chunk

    for b in range(_NBUF):
        pltpu.async_copy(table_hbm.at[idx_v.at[b]], bufs[b], gsems[b])

    @pl.loop(0, n_iter, step=_NBUF)
    def _(g):
        for b in range(_NBUF):
            i = g + b
            pltpu.make_async_copy(
                table_hbm.at[idx_v.at[i]], bufs[b], gsems[b]).wait()
            out_slice = out_hbm.at[pl.ds(base + i * chunk, chunk)]
            pltpu.async_copy(bufs[b], out_slice, ssems[b])
            pltpu.make_async_copy(bufs[b], out_slice, ssems[b]).wait()
            nxt = i + _NBUF

            @pl.when(nxt < n_iter)
            def _():
                pltpu.async_copy(
                    table_hbm.at[idx_v.at[nxt]], bufs[b], gsems[b])


def _sc_gather(idx_flat, table):
    n_rows = idx_flat.shape[0]
    C = table.shape[1]
    rows_per_w = n_rows // _NW
    chunk = 4
    idx3 = idx_flat.reshape(_NW, rows_per_w // chunk, chunk)
    mesh = plsc.VectorSubcoreMesh(
        core_axis_name="c", subcore_axis_name="s",
        num_cores=_NC, num_subcores=_NS)
    body = functools.partial(_sc_gather_body, rows_per_w=rows_per_w,
                             chunk=chunk)

    def wrapped(idx_hbm, table_hbm, out_hbm, *scratch):
        bufs = scratch[:_NBUF]
        gsems = scratch[_NBUF:2 * _NBUF]
        ssems = scratch[2 * _NBUF:3 * _NBUF]
        body(idx_hbm, table_hbm, out_hbm, scratch[3 * _NBUF], bufs, gsems,
             ssems)

    return pl.kernel(
        wrapped,
        out_type=jax.ShapeDtypeStruct((n_rows, C), jnp.float32),
        mesh=mesh,
        scratch_types=(
            [pltpu.VMEM((chunk, C), jnp.float32)] * _NBUF
            + [pltpu.SemaphoreType.DMA] * (2 * _NBUF)
            + [pltpu.VMEM((rows_per_w // chunk, chunk), jnp.int32)]
        ),
    )(idx3, table)


def _loss_body(tgt_ref, rows_ref, loss_ref, acc, *, n_steps, n_rows, R, C):
    step = pl.program_id(0)

    @pl.when(step == 0)
    def _init():
        acc[0] = jnp.float32(0.0)

    rows = rows_ref[...]
    m = jnp.max(rows, axis=1, keepdims=True)
    e = jnp.exp(rows - m)
    s = jnp.sum(e, axis=1, keepdims=True)
    lse = m + jnp.log(s)

    tgt = tgt_ref[0, 0, :]
    lane = lax.broadcasted_iota(jnp.int32, (R, C), 1)
    tv = jnp.sum(jnp.where(lane == tgt[:, None], rows, 0.0), axis=1,
                 keepdims=True)
    acc[0] += jnp.sum(lse - tv)

    @pl.when(step == n_steps - 1)
    def _fin():
        loss_ref[...] = jnp.full((1, 1), acc[0] / jnp.float32(n_rows),
                                 dtype=jnp.float32)


def _tc_loss(logits, targets_flat):
    n_rows, C = logits.shape
    R = 128
    n_steps = n_rows // R
    tgt3 = targets_flat.reshape(n_steps, 1, R)

    loss = pl.pallas_call(
        functools.partial(_loss_body, n_steps=n_steps, n_rows=n_rows, R=R,
                          C=C),
        grid=(n_steps,),
        in_specs=[
            pl.BlockSpec((1, 1, R), lambda i: (i, 0, 0)),
            pl.BlockSpec((R, C), lambda i: (i, 0)),
        ],
        out_specs=pl.BlockSpec((1, 1), lambda i: (0, 0)),
        out_shape=jax.ShapeDtypeStruct((1, 1), jnp.float32),
        scratch_shapes=[pltpu.SMEM((1,), jnp.float32)],
    )(tgt3, logits)
    return loss[0, 0]


def kernel(idx, targets, table):
    B, T = idx.shape
    N = B * T
    idx_flat = idx.reshape(N).astype(jnp.int32)
    tgt_flat = targets.reshape(N).astype(jnp.int32)
    logits = _sc_gather(idx_flat, table)
    loss = _tc_loss(logits, tgt_flat)
    return (logits, loss)
